# Initial kernel scaffold; baseline (speedup 1.0000x reference)
import jax
import jax.numpy as jnp
from jax.experimental import pallas as pl


def kernel(feat, weight, bias, prelu_conv_w, bn_gamma, bn_beta, bn_mean, bn_var, prelu_act_w, edge_weight, graph, diff_graph):
    raise NotImplementedError("write your pallas kernel here")


# speedup vs baseline: 8.5338x; 8.5338x over previous
import functools

import jax
import jax.numpy as jnp
from jax import lax
from jax.experimental import pallas as pl
from jax.experimental.pallas import tpu as pltpu
from jax.experimental.pallas import tpu_sc as plsc

N = 10000
NP = 10240
D = 128
E = 320000
NC = 2
NS = 16
NW = NC * NS
C = 128
CH = 80
EP = NW * CH * C
RPT = NP // NS
NPAD = NP - N

_mesh = plsc.VectorSubcoreMesh(core_axis_name="c", subcore_axis_name="s")


@functools.partial(
    pl.kernel,
    mesh=_mesh,
    out_type=jax.ShapeDtypeStruct((NC * 2 * NP,), jnp.float32),
    scratch_types=[
        pltpu.VMEM((CH, C), jnp.int32),
        pltpu.VMEM((CH, C), jnp.int32),
        pltpu.VMEM((C,), jnp.float32),
        pltpu.VMEM((RPT,), jnp.float32),
        pltpu.VMEM_SHARED((NP,), jnp.float32),
        pltpu.VMEM_SHARED((NP,), jnp.float32),
    ],
)
def _deg_kernel(src_hbm, dst_hbm, out_hbm,
                src_v, dst_v, ones_v, buf_v, acc_out, acc_in):
    c = lax.axis_index("c")
    s = lax.axis_index("s")
    w = s * NC + c
    pltpu.sync_copy(src_hbm.at[w], src_v)
    pltpu.sync_copy(dst_hbm.at[w], dst_v)

    one16 = jnp.ones((16,), jnp.float32)
    zero16 = jnp.zeros((16,), jnp.float32)

    def fill_ones(i, carry):
        ones_v[pl.ds(i * 16, 16)] = one16
        return carry

    lax.fori_loop(0, C // 16, fill_ones, 0)

    def fill_zero(i, carry):
        buf_v[pl.ds(i * 16, 16)] = zero16
        return carry

    lax.fori_loop(0, RPT // 16, fill_zero, 0)
    pltpu.sync_copy(buf_v, acc_out.at[pl.ds(s * RPT, RPT)])
    pltpu.sync_copy(buf_v, acc_in.at[pl.ds(s * RPT, RPT)])
    plsc.subcore_barrier()

    def body(j, carry):
        pltpu.sync_copy(ones_v, acc_out.at[src_v.at[j]], add=True)
        pltpu.sync_copy(ones_v, acc_in.at[dst_v.at[j]], add=True)
        return carry

    lax.fori_loop(0, CH, body, 0)
    plsc.subcore_barrier()
    pltpu.sync_copy(acc_out.at[pl.ds(s * RPT, RPT)], buf_v)
    pltpu.sync_copy(buf_v, out_hbm.at[pl.ds(c * (2 * NP) + s * RPT, RPT)])
    pltpu.sync_copy(acc_in.at[pl.ds(s * RPT, RPT)], buf_v)
    pltpu.sync_copy(buf_v, out_hbm.at[pl.ds(c * (2 * NP) + NP + s * RPT, RPT)])


@functools.partial(
    pl.kernel,
    mesh=_mesh,
    out_type=jax.ShapeDtypeStruct((NC * NP, D), jnp.float32),
    scratch_types=[
        pltpu.VMEM((CH, C), jnp.int32),
        pltpu.VMEM((CH, C), jnp.int32),
        pltpu.VMEM((C, D), jnp.float32),
        pltpu.VMEM_SHARED((NP, D), jnp.float32),
        pltpu.SemaphoreType.DMA,
    ],
)
def _agg_kernel(y_hbm, src_hbm, dst_hbm, out_hbm,
                src_v, dst_v, rows_v, acc_sh, sem):
    c = lax.axis_index("c")
    s = lax.axis_index("s")
    w = s * NC + c
    pltpu.sync_copy(src_hbm.at[w], src_v)
    pltpu.sync_copy(dst_hbm.at[w], dst_v)

    zero16 = jnp.zeros((16,), jnp.float32)

    def fill_zero(i, carry):
        def cols(k, carry2):
            rows_v[i, pl.ds(k * 16, 16)] = zero16
            return carry2
        return lax.fori_loop(0, D // 16, cols, carry)

    lax.fori_loop(0, C, fill_zero, 0)
    for r in range(RPT // C):
        pltpu.sync_copy(rows_v, acc_sh.at[pl.ds(s * RPT + r * C, C)])
    plsc.subcore_barrier()

    def body(j, carry):
        pltpu.async_copy(y_hbm.at[src_v.at[j]], rows_v, sem).wait()
        pltpu.sync_copy(rows_v, acc_sh.at[dst_v.at[j]], add=True)
        return carry

    lax.fori_loop(0, CH, body, 0)
    plsc.subcore_barrier()
    for r in range(RPT // C):
        pltpu.sync_copy(acc_sh.at[pl.ds(s * RPT + r * C, C)], rows_v)
        pltpu.sync_copy(rows_v, out_hbm.at[pl.ds(c * NP + s * RPT + r * C, C)])


def _mm_body(feat_ref, deg_ref, w_ref, y_ref):
    deg = jnp.sum(deg_ref[...], axis=0)
    scale = lax.rsqrt(jnp.maximum(deg, 1.0))
    x = feat_ref[...] * scale[:, None]
    y_ref[...] = jnp.dot(x, w_ref[...], precision=lax.Precision.HIGHEST,
                         preferred_element_type=jnp.float32)


def _epi_body(agg_ref, deg_ref, bias_ref, pcw_ref, g_ref, b_ref, m_ref,
              v_ref, paw_ref, out_ref):
    a = agg_ref[0] + agg_ref[1]
    deg = jnp.sum(deg_ref[...], axis=0)
    a = a * lax.rsqrt(jnp.maximum(deg, 1.0))[:, None]
    h = a + bias_ref[...]
    pcw = pcw_ref[0, 0]
    h = jnp.where(h >= 0, h, pcw * h)
    h = (h - m_ref[...]) * lax.rsqrt(v_ref[...] + 1e-5) * g_ref[...] + b_ref[...]
    paw = paw_ref[0, 0]
    out_ref[...] = jnp.where(h >= 0, h, paw * h)


_BR = 256


def kernel(feat, weight, bias, prelu_conv_w, bn_gamma, bn_beta, bn_mean,
           bn_var, prelu_act_w, edge_weight, graph, diff_graph):
    src = graph[0]
    dst = graph[1]
    pad = EP - E
    pad_idx = N + (jnp.arange(pad, dtype=jnp.int32) % NPAD)
    src_p = jnp.concatenate([src, pad_idx])
    dst_p = jnp.concatenate([dst, pad_idx])
    src_rs = src_p.reshape(NW, CH, C)
    dst_rs = dst_p.reshape(NW, CH, C)
    feat_pad = jnp.pad(feat, ((0, NP - N), (0, 0)))

    degs = _deg_kernel(src_rs, dst_rs).reshape(NC, 2, NP)
    deg_out_p = degs[:, 0]
    deg_in_p = degs[:, 1]

    y = pl.pallas_call(
        _mm_body,
        grid=(NP // _BR,),
        in_specs=[
            pl.BlockSpec((_BR, D), lambda i: (i, 0)),
            pl.BlockSpec((NC, _BR), lambda i: (0, i)),
            pl.BlockSpec((D, D), lambda i: (0, 0)),
        ],
        out_specs=pl.BlockSpec((_BR, D), lambda i: (i, 0)),
        out_shape=jax.ShapeDtypeStruct((NP, D), jnp.float32),
    )(feat_pad, deg_out_p, weight)

    agg_parts = _agg_kernel(y, src_rs, dst_rs).reshape(NC, NP, D)

    vec = lambda a: a.reshape(1, D)
    scalar = lambda a: a.reshape(1, 1)
    h = pl.pallas_call(
        _epi_body,
        grid=(NP // _BR,),
        in_specs=[
            pl.BlockSpec((NC, _BR, D), lambda i: (0, i, 0)),
            pl.BlockSpec((NC, _BR), lambda i: (0, i)),
            pl.BlockSpec((1, D), lambda i: (0, 0)),
            pl.BlockSpec((1, 1), lambda i: (0, 0)),
            pl.BlockSpec((1, D), lambda i: (0, 0)),
            pl.BlockSpec((1, D), lambda i: (0, 0)),
            pl.BlockSpec((1, D), lambda i: (0, 0)),
            pl.BlockSpec((1, D), lambda i: (0, 0)),
            pl.BlockSpec((1, 1), lambda i: (0, 0)),
        ],
        out_specs=pl.BlockSpec((_BR, D), lambda i: (i, 0)),
        out_shape=jax.ShapeDtypeStruct((NP, D), jnp.float32),
    )(agg_parts, deg_in_p, vec(bias), scalar(prelu_conv_w), vec(bn_gamma),
      vec(bn_beta), vec(bn_mean), vec(bn_var), scalar(prelu_act_w))

    return h[:N]
